# Initial kernel scaffold; baseline (speedup 1.0000x reference)
import jax
import jax.numpy as jnp
from jax.experimental import pallas as pl


def kernel(X, W_base_a, W_mean_a, W_logstd_a, W_base_x, W_mean_x, W_logstd_x, Wa, ba, Wx, bx, Ws, bs, edge_index, sen_idx):
    raise NotImplementedError("write your pallas kernel here")


# speedup vs baseline: 3.3613x; 3.3613x over previous
import functools

import jax
import jax.numpy as jnp
from jax import lax
from jax.experimental import pallas as pl
from jax.experimental.pallas import tpu as pltpu
from jax.experimental.pallas import tpu_sc as plsc

_NC = 2
_NS = 16
_B = 80
_H = 128


@functools.partial(jax.jit, static_argnames=("n", "e"))
def _spmm_sc(h2, col4, row3, zrows, *, n, e):
    epw = e // _NS
    ch = epw // _B
    rpw = (n // _NS) // 8 * 8
    rlast = n - rpw * (_NS - 1)
    mesh = plsc.VectorSubcoreMesh(core_axis_name="c", subcore_axis_name="s")

    def body(h_hbm, col_hbm, row_hbm, z_hbm, out_hbm, colv, rowv, buf, acc, sem):
        c = lax.axis_index("c")
        s = lax.axis_index("s")

        @pl.when(s < _NS - 1)
        def _():
            pltpu.sync_copy(z_hbm.at[pl.ds(0, rpw)], acc.at[pl.ds(s * rpw, rpw)])

        @pl.when(s == _NS - 1)
        def _():
            pltpu.sync_copy(z_hbm, acc.at[pl.ds(s * rpw, rlast)])

        pltpu.sync_copy(col_hbm.at[c * _NS + s], colv)
        pltpu.sync_copy(row_hbm.at[s], rowv)
        plsc.subcore_barrier()

        def step(i, carry):
            pltpu.async_copy(h_hbm.at[colv.at[i]], buf, sem).wait()
            pltpu.sync_copy(buf, acc.at[rowv.at[i]], add=True)
            return carry

        lax.fori_loop(0, ch, step, 0)
        plsc.subcore_barrier()

        @pl.when(s < _NS - 1)
        def _():
            pltpu.sync_copy(acc.at[pl.ds(s * rpw, rpw)],
                            out_hbm.at[pl.ds(c * n + s * rpw, rpw)])

        @pl.when(s == _NS - 1)
        def _():
            pltpu.sync_copy(acc.at[pl.ds(s * rpw, rlast)],
                            out_hbm.at[pl.ds(c * n + s * rpw, rlast)])

    return pl.kernel(
        body,
        out_type=jax.ShapeDtypeStruct((2 * n, _H), jnp.float32),
        mesh=mesh,
        scratch_types=[
            pltpu.VMEM((ch, _B), jnp.int32),
            pltpu.VMEM((ch, _B), jnp.int32),
            pltpu.VMEM((_B, _H), jnp.float32),
            pltpu.VMEM_SHARED((n, _H), jnp.float32),
            pltpu.SemaphoreType.DMA,
        ],
    )(h2, col4, row3, zrows)


def _enc_in(x, w2, n, rm):
    nr = n // rm

    def body(x_ref, w_ref, o_ref):
        o_ref[...] = jnp.dot(x_ref[...], w_ref[0],
                             preferred_element_type=jnp.float32)[None]

    return pl.pallas_call(
        body,
        grid=(2, nr),
        in_specs=[
            pl.BlockSpec((rm, x.shape[1]), lambda i, r: (r, 0)),
            pl.BlockSpec((1, w2.shape[1], _H), lambda i, r: (i, 0, 0)),
        ],
        out_specs=pl.BlockSpec((1, rm, _H), lambda i, r: (i, r, 0)),
        out_shape=jax.ShapeDtypeStruct((2, n, _H), jnp.float32),
    )(x, w2)


def _enc_mid(sa2, w2, n, rm):
    nr = n // rm

    def body(x_ref, w_ref, o_ref):
        h = jnp.maximum(x_ref[0], 0.0)
        o_ref[...] = jnp.dot(h, w_ref[0],
                             preferred_element_type=jnp.float32)[None]

    return pl.pallas_call(
        body,
        grid=(2, nr),
        in_specs=[
            pl.BlockSpec((1, rm, _H), lambda i, r: (i, r, 0)),
            pl.BlockSpec((1, _H, _H), lambda i, r: (i, 0, 0)),
        ],
        out_specs=pl.BlockSpec((1, rm, _H), lambda i, r: (i, r, 0)),
        out_shape=jax.ShapeDtypeStruct((2, n, _H), jnp.float32),
    )(sa2, w2)


def _decoder_a(za, s_col, w0, w1, b, n, rm, cn):
    nr = n // rm
    nc = pl.cdiv(n, cn)

    def body(z_ref, s_ref, w0_ref, w1_ref, b_ref, o_ref):
        acc = jnp.dot(z_ref[...], w0_ref[...],
                      preferred_element_type=jnp.float32)
        o_ref[...] = jax.nn.sigmoid(acc + s_ref[...] * w1_ref[...] + b_ref[...])

    return pl.pallas_call(
        body,
        grid=(nr, nc),
        in_specs=[
            pl.BlockSpec((rm, _H), lambda r, c: (r, 0)),
            pl.BlockSpec((rm, 1), lambda r, c: (r, 0)),
            pl.BlockSpec((_H, cn), lambda r, c: (0, c)),
            pl.BlockSpec((1, cn), lambda r, c: (0, c)),
            pl.BlockSpec((1, cn), lambda r, c: (0, c)),
        ],
        out_specs=pl.BlockSpec((rm, cn), lambda r, c: (r, c)),
        out_shape=jax.ShapeDtypeStruct((n, n), jnp.float32),
    )(za, s_col, w0, w1, b)


def _decoder_xs(za, zx, s_col, wx0, wx1, bx, wsa, wsx, bsp, n, rm):
    nr = n // rm
    d = wx0.shape[1]

    def body(za_ref, zx_ref, s_ref, wx0_ref, wx1_ref, bx_ref,
             wsa_ref, wsx_ref, bsp_ref, xp_ref, sg_ref):
        zx = zx_ref[...]
        xp_ref[...] = (jnp.dot(zx, wx0_ref[...], preferred_element_type=jnp.float32)
                       + s_ref[...] * wx1_ref[...] + bx_ref[...])
        logits = (jnp.dot(za_ref[...], wsa_ref[...], preferred_element_type=jnp.float32)
                  + jnp.dot(zx, wsx_ref[...], preferred_element_type=jnp.float32)
                  + bsp_ref[...])
        m = jnp.max(logits, axis=1, keepdims=True)
        ex = jnp.exp(logits - m)
        sg_ref[...] = ex / jnp.sum(ex, axis=1, keepdims=True)

    return pl.pallas_call(
        body,
        grid=(nr,),
        in_specs=[
            pl.BlockSpec((rm, _H), lambda r: (r, 0)),
            pl.BlockSpec((rm, _H), lambda r: (r, 0)),
            pl.BlockSpec((rm, 1), lambda r: (r, 0)),
            pl.BlockSpec((_H, d), lambda r: (0, 0)),
            pl.BlockSpec((1, d), lambda r: (0, 0)),
            pl.BlockSpec((1, d), lambda r: (0, 0)),
            pl.BlockSpec((_H, _H), lambda r: (0, 0)),
            pl.BlockSpec((_H, _H), lambda r: (0, 0)),
            pl.BlockSpec((1, _H), lambda r: (0, 0)),
        ],
        out_specs=[
            pl.BlockSpec((rm, d), lambda r: (r, 0)),
            pl.BlockSpec((rm, _H), lambda r: (r, 0)),
        ],
        out_shape=[
            jax.ShapeDtypeStruct((n, d), jnp.float32),
            jax.ShapeDtypeStruct((n, _H), jnp.float32),
        ],
    )(za, zx, s_col, wx0, wx1, bx, wsa, wsx, bsp)


def kernel(X, W_base_a, W_mean_a, W_logstd_a, W_base_x, W_mean_x, W_logstd_x,
           Wa, ba, Wx, bx, Ws, bs, edge_index, sen_idx):
    n, d = X.shape
    e = edge_index.shape[1]
    rm = 1000

    sen = jnp.asarray(sen_idx, dtype=jnp.int32)
    col_ids = lax.broadcasted_iota(jnp.int32, (1, d), 1)
    X_ns = jnp.where(col_ids == sen, 0.0, X)
    S = jnp.take(X, sen, axis=1).reshape(n, 1)

    row = edge_index[0]
    col = edge_index[1]
    epw = e // _NS
    ch = epw // _B
    col4 = jnp.stack([col, col + n]).reshape(_NC * _NS, ch, _B)
    row3 = row.reshape(_NS, ch, _B)
    zrows = jnp.zeros((n - (n // _NS) // 8 * 8 * (_NS - 1), _H), jnp.float32)

    wb2 = jnp.stack([W_base_a, W_base_x])
    xw2 = _enc_in(X_ns, wb2, n, rm)
    sa2 = _spmm_sc(xw2.reshape(2 * n, _H), col4, row3, zrows, n=n, e=e)
    wm2 = jnp.stack([W_mean_a, W_mean_x])
    hw2 = _enc_mid(sa2.reshape(2, n, _H), wm2, n, rm)
    z2 = _spmm_sc(hw2.reshape(2 * n, _H), col4, row3, zrows, n=n, e=e)
    z_a = z2[:n]
    z_x = z2[n:]

    A_pred = _decoder_a(z_a, S, Wa[:_H], Wa[_H:].reshape(1, n),
                        ba.reshape(1, n), n, rm, 2048)
    ws_pad = jnp.pad(Ws, ((0, 0), (0, _H - Ws.shape[1])))
    bs_pad = jnp.concatenate([bs, jnp.full((_H - bs.shape[0],), -1e30,
                                           jnp.float32)]).reshape(1, _H)
    X_pred, sg = _decoder_xs(z_a, z_x, S, Wx[:_H], Wx[_H:].reshape(1, d),
                             bx.reshape(1, d), ws_pad[:_H], ws_pad[_H:],
                             bs_pad, n, rm)
    S_agg_pred = sg[:, :Ws.shape[1]]
    return (A_pred, X_pred, S_agg_pred)


# speedup vs baseline: 3.9478x; 1.1745x over previous
import functools

import jax
import jax.numpy as jnp
from jax import lax
from jax.experimental import pallas as pl
from jax.experimental.pallas import tpu as pltpu
from jax.experimental.pallas import tpu_sc as plsc

_NC = 2
_NS = 16
_B = 80
_H = 128


@functools.partial(jax.jit, static_argnames=("n", "e"))
def _spmm_sc(h2, col4, row3, zrows, *, n, e):
    epw = e // _NS
    ch = epw // _B
    rpw = (n // _NS) // 8 * 8
    rlast = n - rpw * (_NS - 1)
    mesh = plsc.VectorSubcoreMesh(core_axis_name="c", subcore_axis_name="s")

    nb = 2

    def body(h_hbm, col_hbm, row_hbm, z_hbm, out_hbm, colv, rowv,
             b0, b1, acc, g0, g1, s0, s1):
        bufs = [b0, b1]
        gsems = [g0, g1]
        ssems = [s0, s1]
        c = lax.axis_index("c")
        s = lax.axis_index("s")

        @pl.when(s < _NS - 1)
        def _():
            pltpu.sync_copy(z_hbm.at[pl.ds(0, rpw)], acc.at[pl.ds(s * rpw, rpw)])

        @pl.when(s == _NS - 1)
        def _():
            pltpu.sync_copy(z_hbm, acc.at[pl.ds(s * rpw, rlast)])

        pltpu.sync_copy(col_hbm.at[c * _NS + s], colv)
        pltpu.sync_copy(row_hbm.at[s], rowv)
        plsc.subcore_barrier()

        def pair(i0):
            gds = [pltpu.async_copy(
                h_hbm.at[colv.at[pl.ds((i0 + b) * _B, _B)]], bufs[b], gsems[b])
                for b in range(nb)]
            sds = []
            for b in range(nb):
                gds[b].wait()
                sds.append(pltpu.async_copy(bufs[b], acc.at[rowv.at[i0 + b]],
                                            ssems[b], add=True))
            for d in sds:
                d.wait()

        def step(j, carry):
            pair(j * nb)
            return carry

        lax.fori_loop(0, ch // nb, step, 0)
        for i in range(ch - ch % nb, ch):
            pltpu.async_copy(h_hbm.at[colv.at[pl.ds(i * _B, _B)]], bufs[0],
                             gsems[0]).wait()
            pltpu.async_copy(bufs[0], acc.at[rowv.at[i]], ssems[0],
                             add=True).wait()
        plsc.subcore_barrier()

        @pl.when(s < _NS - 1)
        def _():
            pltpu.sync_copy(acc.at[pl.ds(s * rpw, rpw)],
                            out_hbm.at[pl.ds(c * n + s * rpw, rpw)])

        @pl.when(s == _NS - 1)
        def _():
            pltpu.sync_copy(acc.at[pl.ds(s * rpw, rlast)],
                            out_hbm.at[pl.ds(c * n + s * rpw, rlast)])

    return pl.kernel(
        body,
        out_type=jax.ShapeDtypeStruct((2 * n, _H), jnp.float32),
        mesh=mesh,
        scratch_types=[
            pltpu.VMEM((epw,), jnp.int32),
            pltpu.VMEM((ch, _B), jnp.int32),
        ] + [pltpu.VMEM((_B, _H), jnp.float32) for _ in range(nb)]
          + [pltpu.VMEM_SHARED((n, _H), jnp.float32)]
          + [pltpu.SemaphoreType.DMA for _ in range(2 * nb)],
    )(h2, col4, row3, zrows)


def _enc_in(x, w2, n, rm):
    nr = n // rm

    def body(x_ref, w_ref, o_ref):
        o_ref[...] = jnp.dot(x_ref[...], w_ref[0],
                             preferred_element_type=jnp.float32)[None]

    return pl.pallas_call(
        body,
        grid=(2, nr),
        in_specs=[
            pl.BlockSpec((rm, x.shape[1]), lambda i, r: (r, 0)),
            pl.BlockSpec((1, w2.shape[1], _H), lambda i, r: (i, 0, 0)),
        ],
        out_specs=pl.BlockSpec((1, rm, _H), lambda i, r: (i, r, 0)),
        out_shape=jax.ShapeDtypeStruct((2, n, _H), jnp.float32),
    )(x, w2)


def _enc_mid(sa2, w2, n, rm):
    nr = n // rm

    def body(x_ref, w_ref, o_ref):
        h = jnp.maximum(x_ref[0], 0.0)
        o_ref[...] = jnp.dot(h, w_ref[0],
                             preferred_element_type=jnp.float32)[None]

    return pl.pallas_call(
        body,
        grid=(2, nr),
        in_specs=[
            pl.BlockSpec((1, rm, _H), lambda i, r: (i, r, 0)),
            pl.BlockSpec((1, _H, _H), lambda i, r: (i, 0, 0)),
        ],
        out_specs=pl.BlockSpec((1, rm, _H), lambda i, r: (i, r, 0)),
        out_shape=jax.ShapeDtypeStruct((2, n, _H), jnp.float32),
    )(sa2, w2)


def _decoder_a(za, s_col, w0, w1, b, n, rm, cn):
    nr = n // rm
    nc = pl.cdiv(n, cn)

    def body(z_ref, s_ref, w0_ref, w1_ref, b_ref, o_ref):
        acc = jnp.dot(z_ref[...], w0_ref[...],
                      preferred_element_type=jnp.float32)
        o_ref[...] = jax.nn.sigmoid(acc + s_ref[...] * w1_ref[...] + b_ref[...])

    return pl.pallas_call(
        body,
        grid=(nr, nc),
        in_specs=[
            pl.BlockSpec((rm, _H), lambda r, c: (r, 0)),
            pl.BlockSpec((rm, 1), lambda r, c: (r, 0)),
            pl.BlockSpec((_H, cn), lambda r, c: (0, c)),
            pl.BlockSpec((1, cn), lambda r, c: (0, c)),
            pl.BlockSpec((1, cn), lambda r, c: (0, c)),
        ],
        out_specs=pl.BlockSpec((rm, cn), lambda r, c: (r, c)),
        out_shape=jax.ShapeDtypeStruct((n, n), jnp.float32),
    )(za, s_col, w0, w1, b)


def _decoder_xs(za, zx, s_col, wx0, wx1, bx, wsa, wsx, bsp, n, rm):
    nr = n // rm
    d = wx0.shape[1]

    def body(za_ref, zx_ref, s_ref, wx0_ref, wx1_ref, bx_ref,
             wsa_ref, wsx_ref, bsp_ref, xp_ref, sg_ref):
        zx = zx_ref[...]
        xp_ref[...] = (jnp.dot(zx, wx0_ref[...], preferred_element_type=jnp.float32)
                       + s_ref[...] * wx1_ref[...] + bx_ref[...])
        logits = (jnp.dot(za_ref[...], wsa_ref[...], preferred_element_type=jnp.float32)
                  + jnp.dot(zx, wsx_ref[...], preferred_element_type=jnp.float32)
                  + bsp_ref[...])
        m = jnp.max(logits, axis=1, keepdims=True)
        ex = jnp.exp(logits - m)
        sg_ref[...] = ex / jnp.sum(ex, axis=1, keepdims=True)

    return pl.pallas_call(
        body,
        grid=(nr,),
        in_specs=[
            pl.BlockSpec((rm, _H), lambda r: (r, 0)),
            pl.BlockSpec((rm, _H), lambda r: (r, 0)),
            pl.BlockSpec((rm, 1), lambda r: (r, 0)),
            pl.BlockSpec((_H, d), lambda r: (0, 0)),
            pl.BlockSpec((1, d), lambda r: (0, 0)),
            pl.BlockSpec((1, d), lambda r: (0, 0)),
            pl.BlockSpec((_H, _H), lambda r: (0, 0)),
            pl.BlockSpec((_H, _H), lambda r: (0, 0)),
            pl.BlockSpec((1, _H), lambda r: (0, 0)),
        ],
        out_specs=[
            pl.BlockSpec((rm, d), lambda r: (r, 0)),
            pl.BlockSpec((rm, _H), lambda r: (r, 0)),
        ],
        out_shape=[
            jax.ShapeDtypeStruct((n, d), jnp.float32),
            jax.ShapeDtypeStruct((n, _H), jnp.float32),
        ],
    )(za, zx, s_col, wx0, wx1, bx, wsa, wsx, bsp)


def kernel(X, W_base_a, W_mean_a, W_logstd_a, W_base_x, W_mean_x, W_logstd_x,
           Wa, ba, Wx, bx, Ws, bs, edge_index, sen_idx):
    n, d = X.shape
    e = edge_index.shape[1]
    rm = 1000

    sen = jnp.asarray(sen_idx, dtype=jnp.int32)
    col_ids = lax.broadcasted_iota(jnp.int32, (1, d), 1)
    X_ns = jnp.where(col_ids == sen, 0.0, X)
    S = jnp.take(X, sen, axis=1).reshape(n, 1)

    row = edge_index[0]
    col = edge_index[1]
    epw = e // _NS
    ch = epw // _B
    col4 = jnp.stack([col, col + n]).reshape(_NC * _NS, epw)
    row3 = row.reshape(_NS, ch, _B)
    zrows = jnp.zeros((n - (n // _NS) // 8 * 8 * (_NS - 1), _H), jnp.float32)

    wb2 = jnp.stack([W_base_a, W_base_x])
    xw2 = _enc_in(X_ns, wb2, n, rm)
    sa2 = _spmm_sc(xw2.reshape(2 * n, _H), col4, row3, zrows, n=n, e=e)
    wm2 = jnp.stack([W_mean_a, W_mean_x])
    hw2 = _enc_mid(sa2.reshape(2, n, _H), wm2, n, rm)
    z2 = _spmm_sc(hw2.reshape(2 * n, _H), col4, row3, zrows, n=n, e=e)
    z_a = z2[:n]
    z_x = z2[n:]

    A_pred = _decoder_a(z_a, S, Wa[:_H], Wa[_H:].reshape(1, n),
                        ba.reshape(1, n), n, rm, 2048)
    ws_pad = jnp.pad(Ws, ((0, 0), (0, _H - Ws.shape[1])))
    bs_pad = jnp.concatenate([bs, jnp.full((_H - bs.shape[0],), -1e30,
                                           jnp.float32)]).reshape(1, _H)
    X_pred, sg = _decoder_xs(z_a, z_x, S, Wx[:_H], Wx[_H:].reshape(1, d),
                             bx.reshape(1, d), ws_pad[:_H], ws_pad[_H:],
                             bs_pad, n, rm)
    S_agg_pred = sg[:, :Ws.shape[1]]
    return (A_pred, X_pred, S_agg_pred)
